# Initial kernel scaffold; baseline (speedup 1.0000x reference)
import jax
import jax.numpy as jnp
from jax.experimental import pallas as pl


def kernel(inputs, targets, corrected_targets, features):
    raise NotImplementedError("write your pallas kernel here")


# speedup vs baseline: 1.3061x; 1.3061x over previous
import functools

import jax
import jax.numpy as jnp
from jax.experimental import pallas as pl
from jax.experimental.pallas import tpu as pltpu

B = 1024
D = 64
K = 100000
TEMP = 0.05
CK = 1024
NSTEPS = (K + CK - 1) // CK
K_PAD = NSTEPS * CK


def _loss_kernel(x_ref, f_ref, ct_ref, out_ref, xn_ref, acc_ref, tgt_ref):
    i = pl.program_id(0)

    @pl.when(i == 0)
    def _init():
        x = x_ref[...]
        norm = jnp.sqrt(jnp.sum(x * x, axis=1, keepdims=True))
        xn_ref[...] = x / jnp.maximum(norm, 1e-12)
        acc_ref[...] = jnp.zeros_like(acc_ref)
        tgt_ref[...] = jnp.zeros_like(tgt_ref)

    xn = xn_ref[...]
    f = f_ref[...]
    logits = jax.lax.dot_general(
        xn, f, (((1,), (1,)), ((), ())),
        preferred_element_type=jnp.float32) * (1.0 / TEMP)

    col = i * CK + jax.lax.broadcasted_iota(jnp.int32, (B, CK), 1)
    valid = col < K
    e = jnp.where(valid, jnp.exp(logits), 0.0)
    acc_ref[...] += jnp.sum(e, axis=1, keepdims=True)

    tmask = col == ct_ref[...]
    tgt_ref[...] += jnp.sum(jnp.where(tmask, logits, 0.0), axis=1,
                            keepdims=True)

    @pl.when(i == NSTEPS - 1)
    def _fini():
        logz = jnp.log(acc_ref[...])
        out_ref[...] = jnp.mean(logz - tgt_ref[...]).reshape(1, 1)


@jax.jit
def _run(inputs, corrected_targets, features):
    f_pad = jnp.pad(features, ((0, K_PAD - K), (0, 0)))
    ct = corrected_targets.reshape(B, 1).astype(jnp.int32)
    out = pl.pallas_call(
        _loss_kernel,
        grid=(NSTEPS,),
        in_specs=[
            pl.BlockSpec((B, D), lambda i: (0, 0)),
            pl.BlockSpec((CK, D), lambda i: (i, 0)),
            pl.BlockSpec((B, 1), lambda i: (0, 0)),
        ],
        out_specs=pl.BlockSpec((1, 1), lambda i: (0, 0)),
        out_shape=jax.ShapeDtypeStruct((1, 1), jnp.float32),
        scratch_shapes=[
            pltpu.VMEM((B, D), jnp.float32),
            pltpu.VMEM((B, 1), jnp.float32),
            pltpu.VMEM((B, 1), jnp.float32),
        ],
    )(inputs, f_pad, ct)
    return out[0, 0]


def kernel(inputs, targets, corrected_targets, features):
    del targets
    return _run(inputs, corrected_targets, features)
